# baseline (speedup 1.0000x reference)
import functools

import jax
import jax.numpy as jnp
from jax import lax
from jax.experimental import pallas as pl
from jax.experimental.pallas import tpu as pltpu
from jax.experimental.pallas import tpu_sc as plsc

B = 64
C = 6625
T = 80
MAX_LEN = 25
LPAD = 32
TOTAL_Y = B * MAX_LEN

NTC = 32
NW = 32
NREP = -(-(B - NTC) // NW)
LANES = 16
NT = T // LANES
CCH = 256
NCHUNK = (C + CCH - 1) // CCH
LAST_CH = C - (NCHUNK - 1) * CCH
TAB = 6656


def _sc_body(x_hbm, y_hbm, starts_hbm, lens_hbm, lut_hbm, out_hbm,
             xb_v, y_v, st_v, ln_v, lut_v, nk_v, yk_v, row_v, sem0, sem1):
    cid = lax.axis_index("c")
    sid = lax.axis_index("s")
    w = sid * 2 + cid

    pltpu.sync_copy(y_hbm, y_v)
    pltpu.sync_copy(starts_hbm, st_v)
    pltpu.sync_copy(lens_hbm, ln_v)
    pltpu.sync_copy(lut_hbm, lut_v)

    zero16 = jnp.zeros((LANES,), jnp.float32)

    def _zero(i, carry):
        nk_v[pl.ds(i * LANES, LANES)] = zero16
        yk_v[pl.ds(i * LANES, LANES)] = zero16
        return carry

    lax.fori_loop(0, TAB // LANES, _zero, 0)

    lane_iota = lax.iota(jnp.int32, LANES)
    sems = (sem0, sem1)
    for rep in range(NREP):
        row_v[pl.ds(rep * LANES, LANES)] = zero16

    for rep in range(NREP):
        b = NTC + rep * NW + w

        @pl.when(b < B)
        def _process():
            run_max = [jnp.full((LANES,), -jnp.inf, jnp.float32)
                       for _ in range(NT)]
            run_idx = [jnp.zeros((LANES,), jnp.int32) for _ in range(NT)]

            def chunk_rows(i):
                return LAST_CH if i == NCHUNK - 1 else CCH

            pending = [None, None]
            pending[0] = pltpu.async_copy(
                x_hbm.at[b, pl.ds(0, CCH), :], xb_v.at[0], sems[0])

            for ch in range(NCHUNK):
                slot = ch % 2
                if ch + 1 < NCHUNK:
                    nxt = (ch + 1) % 2
                    rows = chunk_rows(ch + 1)
                    pending[nxt] = pltpu.async_copy(
                        x_hbm.at[b, pl.ds((ch + 1) * CCH, rows), :],
                        xb_v.at[nxt, pl.ds(0, rows), :], sems[nxt])
                pending[slot].wait()

                cbase = ch * CCH

                def class_body(cc, carry, slot=slot, cbase=cbase):
                    ms = list(carry[:NT])
                    ids = list(carry[NT:])
                    cvec = jnp.full((LANES,), cbase + cc, jnp.int32)
                    for k in range(NT):
                        v = xb_v[slot, cc, pl.ds(k * LANES, LANES)]
                        upd = v > ms[k]
                        ms[k] = jnp.where(upd, v, ms[k])
                        ids[k] = jnp.where(upd, cvec, ids[k])
                    return tuple(ms) + tuple(ids)

                carry = tuple(run_max) + tuple(run_idx)
                carry = lax.fori_loop(0, chunk_rows(ch), class_body, carry,
                                      unroll=8)
                run_max[:] = list(carry[:NT])
                run_idx[:] = list(carry[NT:])

            one16 = jnp.full((LANES,), 1.0, jnp.float32)
            true16 = jnp.full((LANES,), True)
            for k in range(NT):
                plsc.addupdate_scatter(nk_v, [run_idx[k]], one16,
                                       mask=true16)

            bvec = jnp.full((LANES,), b, jnp.int32)
            start = plsc.load_gather(st_v, [bvec])
            length = plsc.load_gather(ln_v, [bvec])

            labs, msks = [], []
            for g in range(2):
                off = lane_iota + g * LANES
                msk = off < length
                lab = plsc.load_gather(y_v, [start + off], mask=msk)
                lab = jnp.where(msk, lab, 0)
                labs.append(lab)
                msks.append(msk)
                plsc.addupdate_scatter(yk_v, [lab], one16, mask=msk)

            sum_nk = jnp.float32(0.0)
            ms_g, mult_g = [], []
            for g in range(2):
                mvals = plsc.load_gather(nk_v, [labs[g]], mask=msks[g])
                mults = plsc.load_gather(yk_v, [labs[g]], mask=msks[g])
                ms_g.append(mvals)
                mult_g.append(mults)
                sum_nk = sum_nk + jnp.sum(
                    jnp.where(msks[g], mvals / mults, 0.0))

            log_l = plsc.load_gather(lut_v, [length])
            snk = jnp.full((LANES,), sum_nk, jnp.float32)
            loss = jnp.float32(0.0)
            for g in range(2):
                n_p = jnp.where(snk == 0.0, 1e-5,
                                jnp.maximum(ms_g[g] / snk, 1e-5))
                log_m = plsc.load_gather(
                    lut_v, [mult_g[g].astype(jnp.int32)], mask=msks[g])
                contrib = jnp.where(
                    msks[g], -n_p * (log_m - log_l) / mult_g[g], 0.0)
                loss = loss + jnp.sum(contrib)

            zf = jnp.zeros((LANES,), jnp.float32)
            for k in range(NT):
                plsc.store_scatter(nk_v, [run_idx[k]], zf, mask=true16)
            for g in range(2):
                plsc.store_scatter(yk_v, [labs[g]], zf, mask=msks[g])

            row_v[pl.ds(rep * LANES, LANES)] = jnp.where(
                lane_iota == 0, jnp.full((LANES,), loss), zero16)

    acc = zero16
    for rep in range(NREP):
        acc = acc + row_v[pl.ds(rep * LANES, LANES)]
    row_v[pl.ds(0, LANES)] = acc
    pltpu.sync_copy(row_v.at[pl.ds(0, LANES)], out_hbm.at[w])


def _tc_body(starts_ref, lens_ref, x_ref, y_ref, out_ref):
    b = pl.program_id(0)

    xb = x_ref[0]
    m = jnp.max(xb, axis=0, keepdims=True)
    row_ids = jax.lax.broadcasted_iota(jnp.int32, (C, T), 0)
    cand = jnp.where(xb == m, row_ids, C)
    predicts = jnp.min(cand, axis=0, keepdims=True)

    start = starts_ref[b]
    length = lens_ref[b]

    lab = y_ref[pl.ds(start, LPAD), :]
    pos = jax.lax.broadcasted_iota(jnp.int32, (LPAD, 1), 0)
    valid = pos < length
    lab = jnp.where(valid, lab, -1)

    lab_b = jnp.broadcast_to(lab, (LPAD, LPAD))
    eye = (jax.lax.broadcasted_iota(jnp.int32, (LPAD, LPAD), 0)
           == jax.lax.broadcasted_iota(jnp.int32, (LPAD, LPAD), 1))
    lab_row = jnp.sum(jnp.where(eye, lab_b, 0), axis=0, keepdims=True)

    mult = jnp.sum((lab == lab_row).astype(jnp.float32), axis=1,
                   keepdims=True)
    mcnt = jnp.sum((lab == predicts).astype(jnp.float32), axis=1,
                   keepdims=True)

    validf = valid.astype(jnp.float32)
    inv_mult = validf / mult
    sum_nk = jnp.sum(mcnt * inv_mult, keepdims=True)[:, :1]

    n_p = jnp.where(sum_nk == 0.0, 1e-5, jnp.maximum(mcnt / sum_nk, 1e-5))
    log_yp = jnp.log(mult) - jnp.log(length.astype(jnp.float32))
    contrib = jnp.where(valid, -n_p * log_yp * inv_mult, 0.0)
    loss_b = jnp.sum(contrib, keepdims=True)[:, :1]

    @pl.when(b == 0)
    def _():
        out_ref[...] = jnp.zeros((1, 1), jnp.float32)

    out_ref[...] += loss_b


def _reduce_body(rows_ref, tc_ref, out_ref):
    out_ref[...] = (jnp.sum(rows_ref[...], keepdims=True)
                    + tc_ref[...]) * (1.0 / B)


@jax.jit
def kernel(x, y, target_lengths):
    ends = jnp.cumsum(target_lengths)
    starts = (ends - target_lengths).astype(jnp.int32)
    lens32 = target_lengths.astype(jnp.int32)
    y_pad1 = jnp.zeros((TOTAL_Y + LPAD,), jnp.int32).at[:TOTAL_Y].set(y)
    lut = jnp.log(jnp.maximum(jnp.arange(32, dtype=jnp.float32), 1.0))

    sc_call = functools.partial(
        pl.kernel,
        out_type=jax.ShapeDtypeStruct((NW, LANES), jnp.float32),
        mesh=plsc.VectorSubcoreMesh(core_axis_name="c", subcore_axis_name="s"),
        compiler_params=pltpu.CompilerParams(needs_layout_passes=False,
                                             use_tc_tiling_on_sc=True),
        scratch_types=[
            pltpu.VMEM((2, CCH, T), jnp.float32),
            pltpu.VMEM((TOTAL_Y + LPAD,), jnp.int32),
            pltpu.VMEM((B,), jnp.int32),
            pltpu.VMEM((B,), jnp.int32),
            pltpu.VMEM((32,), jnp.float32),
            pltpu.VMEM((TAB,), jnp.float32),
            pltpu.VMEM((TAB,), jnp.float32),
            pltpu.VMEM((NREP * LANES,), jnp.float32),
            pltpu.SemaphoreType.DMA,
            pltpu.SemaphoreType.DMA,
        ],
    )(_sc_body)
    rows = sc_call(x, y_pad1, starts, lens32, lut)

    y_pad2 = y_pad1.reshape(TOTAL_Y + LPAD, 1)
    tc_part = pl.pallas_call(
        _tc_body,
        grid=(NTC,),
        in_specs=[
            pl.BlockSpec(memory_space=pltpu.SMEM),
            pl.BlockSpec(memory_space=pltpu.SMEM),
            pl.BlockSpec((1, C, T), lambda b: (b, 0, 0)),
            pl.BlockSpec((TOTAL_Y + LPAD, 1), lambda b: (0, 0)),
        ],
        out_specs=pl.BlockSpec((1, 1), lambda b: (0, 0)),
        out_shape=jax.ShapeDtypeStruct((1, 1), jnp.float32),
    )(starts, lens32, x, y_pad2)

    out = pl.pallas_call(
        _reduce_body,
        out_shape=jax.ShapeDtypeStruct((1, 1), jnp.float32),
    )(rows, tc_part)
    return out[0, 0]


# speedup vs baseline: 3.0816x; 3.0816x over previous
import functools

import jax
import jax.numpy as jnp
from jax import lax
from jax.experimental import pallas as pl
from jax.experimental.pallas import tpu as pltpu

B = 64
C = 6625
T = 80
MAX_LEN = 25
LPAD = 32
TOTAL_Y = B * MAX_LEN


def _tc_body(starts_ref, lens_ref, x_ref, y_ref, out_ref):
    b = pl.program_id(0)

    xb = x_ref[0]
    m = jnp.max(xb, axis=1, keepdims=True)
    lane_ids = jax.lax.broadcasted_iota(jnp.int32, (T, C), 1)
    cand = jnp.where(xb == m, lane_ids, C)
    predicts = jnp.min(cand, axis=1, keepdims=True)

    start = starts_ref[b]
    length = lens_ref[b]

    lab = y_ref[pl.ds(start, LPAD), :]
    pos = jax.lax.broadcasted_iota(jnp.int32, (LPAD, 1), 0)
    valid_col = pos < length
    lab = jnp.where(valid_col, lab, -1)

    lab_b = jnp.broadcast_to(lab, (LPAD, LPAD))
    eye = (jax.lax.broadcasted_iota(jnp.int32, (LPAD, LPAD), 0)
           == jax.lax.broadcasted_iota(jnp.int32, (LPAD, LPAD), 1))
    lab_row = jnp.sum(jnp.where(eye, lab_b, 0), axis=0, keepdims=True)

    mult = jnp.sum((lab == lab_row).astype(jnp.float32), axis=0,
                   keepdims=True)
    mcnt = jnp.sum((predicts == lab_row).astype(jnp.float32), axis=0,
                   keepdims=True)

    valid_row = (jax.lax.broadcasted_iota(jnp.int32, (1, LPAD), 1)
                 < length)
    validf = valid_row.astype(jnp.float32)
    inv_mult = validf / mult
    sum_nk = jnp.sum(mcnt * inv_mult, keepdims=True)[:, :1]

    n_p = jnp.where(sum_nk == 0.0, 1e-5, jnp.maximum(mcnt / sum_nk, 1e-5))
    log_yp = jnp.log(mult) - jnp.log(length.astype(jnp.float32))
    contrib = jnp.where(valid_row, -n_p * log_yp * inv_mult, 0.0)
    loss_b = jnp.sum(contrib, keepdims=True)[:, :1]

    @pl.when(b == 0)
    def _():
        out_ref[...] = jnp.zeros((1, 1), jnp.float32)

    out_ref[...] += loss_b * (1.0 / B)


@jax.jit
def kernel(x, y, target_lengths):
    ends = jnp.cumsum(target_lengths)
    starts = (ends - target_lengths).astype(jnp.int32)
    lens32 = target_lengths.astype(jnp.int32)
    y_pad = jnp.zeros((TOTAL_Y + LPAD, 1), jnp.int32).at[:TOTAL_Y, 0].set(y)
    xt = jnp.swapaxes(x, 1, 2)

    out = pl.pallas_call(
        _tc_body,
        grid=(B,),
        in_specs=[
            pl.BlockSpec(memory_space=pltpu.SMEM),
            pl.BlockSpec(memory_space=pltpu.SMEM),
            pl.BlockSpec((1, T, C), lambda b: (b, 0, 0)),
            pl.BlockSpec((TOTAL_Y + LPAD, 1), lambda b: (0, 0)),
        ],
        out_specs=pl.BlockSpec((1, 1), lambda b: (0, 0)),
        out_shape=jax.ShapeDtypeStruct((1, 1), jnp.float32),
    )(starts, lens32, xt, y_pad)
    return out[0, 0]
